# Initial kernel scaffold; baseline (speedup 1.0000x reference)
import jax
import jax.numpy as jnp
from jax.experimental import pallas as pl


def kernel(x, edge_index, edge_kernel, W1, b1, a1, W2, b2, a2):
    raise NotImplementedError("write your pallas kernel here")


# speedup vs baseline: 2.5975x; 2.5975x over previous
import functools

import jax
import jax.numpy as jnp
from jax import lax
from jax.experimental import pallas as pl
from jax.experimental.pallas import tpu as pltpu
from jax.experimental.pallas import tpu_sc as plsc

N = 10000
E = 320000
CH = 128
K = 27

NC = 2
NS = 16
NW = NC * NS
E_PER_W = E // NW
CHUNK = 125
NCHUNK = E_PER_W // CHUNK
ROWS_PER_TILE = 640
NP = NS * ROWS_PER_TILE


def _gidx_body(src_ref, kern_ref, out_ref):
    out_ref[...] = src_ref[...] * K + kern_ref[...]


def _compute_gidx(src, kern):
    src2 = src.reshape(E // CH, CH)
    kern2 = kern.reshape(E // CH, CH)
    out = pl.pallas_call(
        _gidx_body,
        out_shape=jax.ShapeDtypeStruct((E // CH, CH), jnp.int32),
    )(src2, kern2)
    return out.reshape(E)


def _matmul_body(x_ref, w_ref, out_ref):
    out_ref[...] = jnp.dot(x_ref[...], w_ref[...],
                           preferred_element_type=jnp.float32)


def _node_transform(x, w_r, bm=1000):
    grid = N // bm
    return pl.pallas_call(
        _matmul_body,
        grid=(grid,),
        in_specs=[
            pl.BlockSpec((bm, CH), lambda i: (i, 0)),
            pl.BlockSpec((CH, K * CH), lambda i: (0, 0)),
        ],
        out_specs=pl.BlockSpec((bm, K * CH), lambda i: (i, 0)),
        out_shape=jax.ShapeDtypeStruct((N, K * CH), jnp.float32),
    )(x, w_r)


def _combine_matmul_body(p0_ref, p1_ref, b_ref, a_ref, w_ref, out_ref):
    t = p0_ref[...] + p1_ref[...] + b_ref[...]
    y = jnp.where(t >= 0, t, a_ref[0, 0] * t)
    out_ref[...] = jnp.dot(y, w_ref[...], preferred_element_type=jnp.float32)


def _combine_and_transform(p0, p1, b, a, w_r, bm=1000):
    grid = N // bm
    return pl.pallas_call(
        _combine_matmul_body,
        grid=(grid,),
        in_specs=[
            pl.BlockSpec((bm, CH), lambda i: (i, 0)),
            pl.BlockSpec((bm, CH), lambda i: (i, 0)),
            pl.BlockSpec((1, CH), lambda i: (0, 0)),
            pl.BlockSpec((1, 1), lambda i: (0, 0)),
            pl.BlockSpec((CH, K * CH), lambda i: (0, 0)),
        ],
        out_specs=pl.BlockSpec((bm, K * CH), lambda i: (i, 0)),
        out_shape=jax.ShapeDtypeStruct((N, K * CH), jnp.float32),
    )(p0, p1, b, a, w_r)


def _final_body(p0_ref, p1_ref, x_ref, b_ref, a_ref, out_ref):
    t = p0_ref[...] + p1_ref[...] + b_ref[...] + x_ref[...]
    out_ref[...] = jnp.where(t >= 0, t, a_ref[0, 0] * t)


def _final(p0, p1, x, b, a, bm=1000):
    grid = N // bm
    return pl.pallas_call(
        _final_body,
        grid=(grid,),
        in_specs=[
            pl.BlockSpec((bm, CH), lambda i: (i, 0)),
            pl.BlockSpec((bm, CH), lambda i: (i, 0)),
            pl.BlockSpec((bm, CH), lambda i: (i, 0)),
            pl.BlockSpec((1, CH), lambda i: (0, 0)),
            pl.BlockSpec((1, 1), lambda i: (0, 0)),
        ],
        out_specs=pl.BlockSpec((bm, CH), lambda i: (i, 0)),
        out_shape=jax.ShapeDtypeStruct((N, CH), jnp.float32),
    )(p0, p1, x, b, a)


_SC_MESH = plsc.VectorSubcoreMesh(core_axis_name="c", subcore_axis_name="s")


@functools.partial(
    pl.kernel,
    out_type=jax.ShapeDtypeStruct((NC, NP, CH), jnp.float32),
    mesh=_SC_MESH,
    scratch_types=[
        pltpu.VMEM_SHARED((NP, CH), jnp.float32),
        pltpu.VMEM((NCHUNK, CHUNK), jnp.int32),
        pltpu.VMEM((NCHUNK, CHUNK), jnp.int32),
        pltpu.VMEM((CHUNK, CH), jnp.float32),
        pltpu.SemaphoreType.DMA,
    ],
)
def _sc_gather_scatter(h_hbm, gidx_hbm, dst_hbm, zeros_hbm, out_hbm,
                       acc, gidx_v, dst_v, rows_v, sem):
    c = lax.axis_index("c")
    s = lax.axis_index("s")
    wid = c * NS + s

    pltpu.sync_copy(zeros_hbm, acc.at[pl.ds(s * ROWS_PER_TILE, ROWS_PER_TILE)])
    pltpu.sync_copy(gidx_hbm.at[wid], gidx_v)
    pltpu.sync_copy(dst_hbm.at[wid], dst_v)
    plsc.subcore_barrier()

    def chunk(j, carry):
        pltpu.async_copy(h_hbm.at[gidx_v.at[j]], rows_v, sem).wait()
        pltpu.sync_copy(rows_v, acc.at[dst_v.at[j]], add=True)
        return carry

    lax.fori_loop(0, NCHUNK, chunk, 0, unroll=False)

    plsc.subcore_barrier()
    pltpu.sync_copy(acc.at[pl.ds(s * ROWS_PER_TILE, ROWS_PER_TILE)],
                    out_hbm.at[c, pl.ds(s * ROWS_PER_TILE, ROWS_PER_TILE)])


def kernel(x, edge_index, edge_kernel, W1, b1, a1, W2, b2, a2):
    src = edge_index[0]
    dst = edge_index[1]

    gidx = _compute_gidx(src, edge_kernel)
    gidx3 = gidx.reshape(NW, NCHUNK, CHUNK)
    dst3 = dst.reshape(NW, NCHUNK, CHUNK)
    zeros = jnp.zeros((ROWS_PER_TILE, CH), jnp.float32)

    w1_r = jnp.transpose(W1, (1, 0, 2)).reshape(CH, K * CH)
    w2_r = jnp.transpose(W2, (1, 0, 2)).reshape(CH, K * CH)
    b1_2 = b1.reshape(1, CH)
    b2_2 = b2.reshape(1, CH)
    a1_2 = a1.reshape(1, 1)
    a2_2 = a2.reshape(1, 1)

    h1 = _node_transform(x, w1_r).reshape(N * K, CH)
    p1 = _sc_gather_scatter(h1, gidx3, dst3, zeros)
    h2 = _combine_and_transform(p1[0], p1[1], b1_2, a1_2, w2_r)
    h2 = h2.reshape(N * K, CH)
    p2 = _sc_gather_scatter(h2, gidx3, dst3, zeros)
    return _final(p2[0], p2[1], x, b2_2, a2_2)


# speedup vs baseline: 2.9264x; 1.1266x over previous
import functools

import jax
import jax.numpy as jnp
from jax import lax
from jax.experimental import pallas as pl
from jax.experimental.pallas import tpu as pltpu
from jax.experimental.pallas import tpu_sc as plsc

N = 10000
E = 320000
CH = 128
K = 27

NC = 2
NS = 16
NW = NC * NS
E_PER_W = E // NW
CHUNK = 125
NCHUNK = E_PER_W // CHUNK
ROWS_PER_TILE = 640
NP = NS * ROWS_PER_TILE


def _gidx_body(src_ref, kern_ref, out_ref):
    out_ref[...] = src_ref[...] * K + kern_ref[...]


def _compute_gidx(src, kern):
    src2 = src.reshape(E // CH, CH)
    kern2 = kern.reshape(E // CH, CH)
    out = pl.pallas_call(
        _gidx_body,
        out_shape=jax.ShapeDtypeStruct((E // CH, CH), jnp.int32),
    )(src2, kern2)
    return out.reshape(E)


def _matmul_body(x_ref, w_ref, out_ref):
    out_ref[...] = jnp.dot(x_ref[...], w_ref[...],
                           preferred_element_type=jnp.float32)


def _node_transform(x, w_r, bm=1000):
    grid = N // bm
    return pl.pallas_call(
        _matmul_body,
        grid=(grid,),
        in_specs=[
            pl.BlockSpec((bm, CH), lambda i: (i, 0)),
            pl.BlockSpec((CH, K * CH), lambda i: (0, 0)),
        ],
        out_specs=pl.BlockSpec((bm, K * CH), lambda i: (i, 0)),
        out_shape=jax.ShapeDtypeStruct((N, K * CH), jnp.float32),
    )(x, w_r)


def _combine_matmul_body(p0_ref, p1_ref, b_ref, a_ref, w_ref, out_ref):
    t = p0_ref[...] + p1_ref[...] + b_ref[...]
    y = jnp.where(t >= 0, t, a_ref[0, 0] * t)
    out_ref[...] = jnp.dot(y, w_ref[...], preferred_element_type=jnp.float32)


def _combine_and_transform(p0, p1, b, a, w_r, bm=1000):
    grid = N // bm
    return pl.pallas_call(
        _combine_matmul_body,
        grid=(grid,),
        in_specs=[
            pl.BlockSpec((bm, CH), lambda i: (i, 0)),
            pl.BlockSpec((bm, CH), lambda i: (i, 0)),
            pl.BlockSpec((1, CH), lambda i: (0, 0)),
            pl.BlockSpec((1, 1), lambda i: (0, 0)),
            pl.BlockSpec((CH, K * CH), lambda i: (0, 0)),
        ],
        out_specs=pl.BlockSpec((bm, K * CH), lambda i: (i, 0)),
        out_shape=jax.ShapeDtypeStruct((N, K * CH), jnp.float32),
    )(p0, p1, b, a, w_r)


def _final_body(p0_ref, p1_ref, x_ref, b_ref, a_ref, out_ref):
    t = p0_ref[...] + p1_ref[...] + b_ref[...] + x_ref[...]
    out_ref[...] = jnp.where(t >= 0, t, a_ref[0, 0] * t)


def _final(p0, p1, x, b, a, bm=1000):
    grid = N // bm
    return pl.pallas_call(
        _final_body,
        grid=(grid,),
        in_specs=[
            pl.BlockSpec((bm, CH), lambda i: (i, 0)),
            pl.BlockSpec((bm, CH), lambda i: (i, 0)),
            pl.BlockSpec((bm, CH), lambda i: (i, 0)),
            pl.BlockSpec((1, CH), lambda i: (0, 0)),
            pl.BlockSpec((1, 1), lambda i: (0, 0)),
        ],
        out_specs=pl.BlockSpec((bm, CH), lambda i: (i, 0)),
        out_shape=jax.ShapeDtypeStruct((N, CH), jnp.float32),
    )(p0, p1, x, b, a)


_SC_MESH = plsc.VectorSubcoreMesh(core_axis_name="c", subcore_axis_name="s")


@functools.partial(
    pl.kernel,
    out_type=jax.ShapeDtypeStruct((NC, NP, CH), jnp.float32),
    mesh=_SC_MESH,
    scratch_types=[
        pltpu.VMEM_SHARED((NP, CH), jnp.float32),
        pltpu.VMEM((NCHUNK // 2, CHUNK), jnp.int32),
        pltpu.VMEM((NCHUNK // 2, CHUNK), jnp.int32),
        pltpu.VMEM((2, CHUNK, CH), jnp.float32),
        pltpu.SemaphoreType.DMA,
        pltpu.SemaphoreType.DMA,
        pltpu.SemaphoreType.DMA,
        pltpu.SemaphoreType.DMA,
    ],
)
def _sc_gather_scatter(h_hbm, gidx_hbm, dst_hbm, zeros_hbm, out_hbm,
                       acc, gidx_v, dst_v, rows_v, g0, g1, a0, a1):
    c = lax.axis_index("c")
    s = lax.axis_index("s")
    wid = c * NS + s
    gsem = (g0, g1)
    asem = (a0, a1)
    HALF = NCHUNK // 2

    pltpu.sync_copy(zeros_hbm, acc.at[pl.ds(s * ROWS_PER_TILE, ROWS_PER_TILE)])
    plsc.subcore_barrier()

    for half in range(2):
        pltpu.sync_copy(gidx_hbm.at[wid, pl.ds(half * HALF, HALF)], gidx_v)
        pltpu.sync_copy(dst_hbm.at[wid, pl.ds(half * HALF, HALF)], dst_v)
        pltpu.async_copy(h_hbm.at[gidx_v.at[0]], rows_v.at[0], gsem[0])

        def group(g, carry):
            for i in range(2):
                j = g * 2 + i
                o = (i + 1) % 2
                pltpu.make_async_copy(h_hbm.at[gidx_v.at[j]],
                                      rows_v.at[i], gsem[i]).wait()
                pltpu.async_copy(rows_v.at[i], acc.at[dst_v.at[j]], asem[i],
                                 add=True)

                @pl.when(j + 1 < HALF)
                def _():
                    @pl.when(j >= 1)
                    def _():
                        pltpu.make_async_copy(
                            rows_v.at[o], acc.at[dst_v.at[j - 1]],
                            asem[o]).wait()

                    pltpu.async_copy(h_hbm.at[gidx_v.at[j + 1]],
                                     rows_v.at[o], gsem[o])
            return carry

        lax.fori_loop(0, HALF // 2, group, 0, unroll=False)

        pltpu.make_async_copy(rows_v.at[0], acc.at[dst_v.at[HALF - 2]],
                              asem[0]).wait()
        pltpu.make_async_copy(rows_v.at[1], acc.at[dst_v.at[HALF - 1]],
                              asem[1]).wait()

    plsc.subcore_barrier()
    pltpu.sync_copy(acc.at[pl.ds(s * ROWS_PER_TILE, ROWS_PER_TILE)],
                    out_hbm.at[c, pl.ds(s * ROWS_PER_TILE, ROWS_PER_TILE)])


def kernel(x, edge_index, edge_kernel, W1, b1, a1, W2, b2, a2):
    src = edge_index[0]
    dst = edge_index[1]

    gidx = _compute_gidx(src, edge_kernel)
    gidx3 = gidx.reshape(NW, NCHUNK, CHUNK)
    dst3 = dst.reshape(NW, NCHUNK, CHUNK)
    zeros = jnp.zeros((ROWS_PER_TILE, CH), jnp.float32)

    w1_r = jnp.transpose(W1, (1, 0, 2)).reshape(CH, K * CH)
    w2_r = jnp.transpose(W2, (1, 0, 2)).reshape(CH, K * CH)
    b1_2 = b1.reshape(1, CH)
    b2_2 = b2.reshape(1, CH)
    a1_2 = a1.reshape(1, 1)
    a2_2 = a2.reshape(1, 1)

    h1 = _node_transform(x, w1_r).reshape(N * K, CH)
    p1 = _sc_gather_scatter(h1, gidx3, dst3, zeros)
    h2 = _combine_and_transform(p1[0], p1[1], b1_2, a1_2, w2_r)
    h2 = h2.reshape(N * K, CH)
    p2 = _sc_gather_scatter(h2, gidx3, dst3, zeros)
    return _final(p2[0], p2[1], x, b2_2, a2_2)
